# Initial kernel scaffold; baseline (speedup 1.0000x reference)
import jax
import jax.numpy as jnp
from jax.experimental import pallas as pl


def kernel(layer_in, weight):
    raise NotImplementedError("write your pallas kernel here")


# speedup vs baseline: 4.5763x; 4.5763x over previous
import functools

import jax
import jax.numpy as jnp
from jax import lax
from jax.experimental import pallas as pl
from jax.experimental.pallas import tpu as pltpu
from jax.experimental.pallas import tpu_sc as plsc

MAX_SPIKE_TIME = 100000.0
B = 512
IN_SIZE = 512
OUT_SIZE = 128
L = 16
NCHUNK = IN_SIZE // L
OCHUNK = OUT_SIZE // L
NW = 32
ROWS_PER_W = B // NW
CSZ = 128
NBIG = IN_SIZE // CSZ
KC_PER_BIG = CSZ // L

_GATHER_DNUMS = lax.GatherDimensionNumbers(
    offset_dims=(), collapsed_slice_dims=(0,), start_index_map=(0,))


def _lane_bcast(v, lane):
    idx = jnp.full((L,), lane, jnp.int32)
    return lax.gather(v, idx[:, None], dimension_numbers=_GATHER_DNUMS,
                      slice_sizes=(1,),
                      mode=lax.GatherScatterMode.PROMISE_IN_BOUNDS)


def _snn_body(x_hbm, w_hbm, out_hbm, w_v, x_v, sx_v, sxn_v, sidx_v, orow_v):
    wid = lax.axis_index("s") * 2 + lax.axis_index("c")
    pltpu.sync_copy(w_hbm, w_v)
    iota = lax.iota(jnp.int32, L)

    def row_body(r, _):
        row = wid * ROWS_PER_W + r
        pltpu.sync_copy(x_hbm.at[row], x_v)

        sxn_v[pl.ds(IN_SIZE - L, L)] = jnp.full((L,), MAX_SPIKE_TIME,
                                                jnp.float32)

        def ichunk_body(ic, _):
            xi = x_v[pl.ds(ic * L, L)]
            ii = iota + ic * L

            def j_le(jc, rank):
                xj = x_v[pl.ds(jc * L, L)]
                for l in range(L):
                    xjl = _lane_bcast(xj, l)
                    rank = rank + (xjl <= xi).astype(jnp.int32)
                return rank

            def j_lt(jc, rank):
                xj = x_v[pl.ds(jc * L, L)]
                for l in range(L):
                    xjl = _lane_bcast(xj, l)
                    rank = rank + (xjl < xi).astype(jnp.int32)
                return rank

            rank = lax.fori_loop(0, ic, j_le, jnp.zeros((L,), jnp.int32))
            rank = lax.fori_loop(ic + 1, NCHUNK, j_lt, rank)
            for l in range(L):
                xjl = _lane_bcast(xi, l)
                cond = (xjl < xi) | ((xjl == xi) & (iota > l))
                rank = rank + cond.astype(jnp.int32)
            plsc.store_scatter(sx_v, [rank], xi)
            plsc.store_scatter(sidx_v, [rank], ii)
            plsc.store_scatter(sxn_v, [rank - 1], xi, mask=rank >= 1)
            return 0

        lax.fori_loop(0, NCHUNK, ichunk_body, 0)

        mn = [jnp.full((L,), MAX_SPIKE_TIME, jnp.float32)] * OCHUNK
        carryw = [jnp.zeros((L,), jnp.float32)] * OCHUNK
        carrywi = [jnp.zeros((L,), jnp.float32)] * OCHUNK
        for c in range(NBIG):
            cw_l = carryw
            cwi_l = carrywi

            def kchunk_body(kc, carry, _cw=cw_l, _cwi=cwi_l, _c=c):
                cw = list(carry[0:OCHUNK])
                cwi = list(carry[OCHUNK:2 * OCHUNK])
                mnl = list(carry[2 * OCHUNK:3 * OCHUNK])
                base = _c * CSZ + kc * L
                adr_c = sidx_v[pl.ds(base, L)]
                sx_c = sx_v[pl.ds(base, L)]
                sxn_c = sxn_v[pl.ds(base, L)]
                for l in range(L):
                    ridx = _lane_bcast(adr_c, l)
                    xk = _lane_bcast(sx_c, l)
                    xn = _lane_bcast(sxn_c, l)
                    for o in range(OCHUNK):
                        wv = plsc.load_gather(w_v, [ridx, iota + o * L])
                        cw[o] = cw[o] + wv
                        cwi[o] = cwi[o] + wv * xk
                        wsum = cw[o] + _cw[o]
                        wisum = cwi[o] + _cwi[o]
                        den = jnp.maximum(wsum - 1.0, 1e-10)
                        t = wisum / den
                        t = jnp.where(wsum < 1.0, MAX_SPIKE_TIME, t)
                        t = jnp.where(t < xk, MAX_SPIKE_TIME, t)
                        t = jnp.where(t > xn, MAX_SPIKE_TIME, t)
                        mnl[o] = jnp.minimum(mnl[o], t)
                return tuple(cw) + tuple(cwi) + tuple(mnl)

            zero = jnp.zeros((L,), jnp.float32)
            init = (zero,) * (2 * OCHUNK) + tuple(mn)
            res = lax.fori_loop(0, KC_PER_BIG, kchunk_body, init)
            mn = list(res[2 * OCHUNK:3 * OCHUNK])
            carryw = [carryw[o] + res[o] for o in range(OCHUNK)]
            carrywi = [carrywi[o] + res[OCHUNK + o] for o in range(OCHUNK)]

        for o in range(OCHUNK):
            orow_v[pl.ds(o * L, L)] = mn[o]
        pltpu.sync_copy(orow_v, out_hbm.at[row])
        return 0

    lax.fori_loop(0, ROWS_PER_W, row_body, 0)


@jax.jit
def kernel(layer_in, weight):
    mesh = plsc.VectorSubcoreMesh(core_axis_name="c", subcore_axis_name="s")
    f = pl.kernel(
        _snn_body,
        out_type=jax.ShapeDtypeStruct((B, OUT_SIZE), jnp.float32),
        mesh=mesh,
        scratch_types=[
            pltpu.VMEM((IN_SIZE, OUT_SIZE), jnp.float32),
            pltpu.VMEM((IN_SIZE,), jnp.float32),
            pltpu.VMEM((IN_SIZE,), jnp.float32),
            pltpu.VMEM((IN_SIZE,), jnp.float32),
            pltpu.VMEM((IN_SIZE,), jnp.int32),
            pltpu.VMEM((OUT_SIZE,), jnp.float32),
        ],
        compiler_params=pltpu.CompilerParams(needs_layout_passes=False),
    )
    return f(layer_in, weight)
